# baseline (speedup 1.0000x reference)
import jax
import jax.numpy as jnp
from jax import lax
from jax.experimental import pallas as pl
from jax.experimental.pallas import tpu as pltpu
from jax.experimental.pallas import tpu_sc as plsc

NU = 10000
NI = 40000
NN = 50000
H = 64
HH = H // 2
ET = 800000
BS = 4096


_BM = 200


def _mm_body(s_ref, x_ref, o_ref):
    o_ref[...] = jnp.dot(s_ref[...], x_ref[...],
                         preferred_element_type=jnp.float32)


def _matmul(S, x):
    m, k = S.shape
    n = x.shape[1]
    return pl.pallas_call(
        _mm_body,
        grid=(m // _BM,),
        in_specs=[pl.BlockSpec((_BM, k), lambda i: (i, 0)),
                  pl.BlockSpec((k, n), lambda i: (0, 0))],
        out_specs=pl.BlockSpec((_BM, n), lambda i: (i, 0)),
        out_shape=jax.ShapeDtypeStruct((m, n), jnp.float32),
        compiler_params=pltpu.CompilerParams(
            dimension_semantics=("arbitrary",)),
    )(S, x)


_CK = 384
_SUB = 128
_NSUB = _CK // _SUB
_TCH = 132
_EPAD = 16 * _TCH * _CK
_ACC_ROWS = 50176
_ZR = _ACC_ROWS // 16
_WBR = NN // 16


def _hop_body(zeros_h, ein_h, src_h, dst3_h, w_h, eout_h,
              sv, dv, sdv, wv, rv, acc, isem0, isem1, gsem, ssem):
    c = lax.axis_index("c")
    s = lax.axis_index("s")
    shift = c * NN
    c0 = s * _TCH
    isems = (isem0, isem1)

    pltpu.sync_copy(zeros_h, acc.at[pl.ds(s * _ZR, _ZR)])
    plsc.subcore_barrier()

    def idx_copies(cid, b):
        base = (c0 + cid) * _CK
        return (
            pltpu.make_async_copy(src_h.at[pl.ds(base, _CK)], sv.at[b],
                                  isems[b]),
            pltpu.make_async_copy(dst3_h.at[c0 + cid], dv.at[b], isems[b]),
            pltpu.make_async_copy(w_h.at[pl.ds(base, _CK)], wv.at[b],
                                  isems[b]),
        )

    def shift_src(b):
        def sb(g, _):
            sl = pl.ds(g * 16, 16)
            sv[b, sl] = sv[b, sl] + shift
            return 0
        lax.fori_loop(0, _CK // 16, sb, 0)

    def gather_copies(b):
        return [pltpu.make_async_copy(
            ein_h.at[sv.at[b, pl.ds(j * _SUB, _SUB)]],
            rv.at[b, pl.ds(j * _SUB, _SUB)], gsem) for j in range(_NSUB)]

    def scatter_copies(b):
        return [pltpu.make_async_copy(
            rv.at[b, pl.ds(j * _SUB, _SUB)], acc.at[sdv.at[b, j]], ssem)
            for j in range(_NSUB)]

    def compute(b):
        def grp(g, _):
            wvec = wv[b, pl.ds(g * 16, 16)]
            for l in range(16):
                wj = wvec[l]
                for f in range(2):
                    fs = pl.ds(f * 16, 16)
                    rv[b, g * 16 + l, fs] = rv[b, g * 16 + l, fs] * wj
            return 0

        lax.fori_loop(0, _CK // 16, grp, 0, unroll=2)
        for j in range(_NSUB):
            def cpy(g, _):
                sl = pl.ds(g * 16, 16)
                sdv[b, j, sl] = dv[b, j, sl]
                return 0
            lax.fori_loop(0, _SUB // 16, cpy, 0)
        for d in scatter_copies(b):
            d.start(add=True)

    def step(cid, b):
        nb = 1 - b
        for d in gather_copies(b):
            d.wait()

        @pl.when(cid + 1 < _TCH)
        def _():
            for d in idx_copies(cid + 1, nb):
                d.wait()
            shift_src(nb)

        @pl.when(cid >= 1)
        def _():
            for d in scatter_copies(nb):
                d.wait()

        @pl.when(cid + 1 < _TCH)
        def _():
            for d in gather_copies(nb):
                d.start()

        compute(b)

        @pl.when(cid + 2 < _TCH)
        def _():
            for d in idx_copies(cid + 2, b):
                d.start()

    for d in idx_copies(0, 0):
        d.start()
    for d in idx_copies(0, 0):
        d.wait()
    shift_src(0)
    for d in gather_copies(0):
        d.start()
    for d in idx_copies(1, 1):
        d.start()

    def pair(i2, _):
        step(2 * i2, 0)
        step(2 * i2 + 1, 1)
        return 0

    lax.fori_loop(0, _TCH // 2, pair, 0)
    for d in scatter_copies(1):
        d.wait()

    plsc.subcore_barrier()

    pltpu.sync_copy(acc.at[pl.ds(s * _WBR, _WBR)],
                    eout_h.at[pl.ds(c * NN + s * _WBR, _WBR)])


def _hop(zeros, ein, src, dst3, w):
    mesh = plsc.VectorSubcoreMesh(core_axis_name="c", subcore_axis_name="s")
    return pl.kernel(
        _hop_body,
        out_type=jax.ShapeDtypeStruct((2 * NN, HH), jnp.float32),
        mesh=mesh,
        scratch_types=[
            pltpu.VMEM((2, _CK), jnp.int32),
            pltpu.VMEM((2, _NSUB, _SUB), jnp.int32),
            pltpu.VMEM((2, _NSUB, _SUB), jnp.int32),
            pltpu.VMEM((2, _CK), jnp.float32),
            pltpu.VMEM((2, _CK, HH), jnp.float32),
            pltpu.VMEM_SHARED((_ACC_ROWS, HH), jnp.float32),
            pltpu.SemaphoreType.DMA,
            pltpu.SemaphoreType.DMA,
            pltpu.SemaphoreType.DMA,
            pltpu.SemaphoreType.DMA,
        ],
        compiler_params=pltpu.CompilerParams(use_tc_tiling_on_sc=False),
    )(zeros, ein, src, dst3, w)


_BPW = BS // 32


def _final_body(users_h, pos_h, neg_h, u1_h, e1s_h, e2s_h, it1_h,
                r0_h, r1_h, r2_h,
                us_o, ps_o, ns_o, ura_o, urb_o, pra_o, prb_o, nra_o, nrb_o,
                idxv, rows_a, rows_b, rows_c, ha, hb, hc, sem):
    c = lax.axis_index("c")
    s = lax.axis_index("s")
    w = s * 2 + c
    base = w * _BPW
    sl_out = pl.ds(base, _BPW)

    def shift_idx(off):
        for g in range(_BPW // 16):
            sl = pl.ds(g * 16, 16)
            idxv[sl] = idxv[sl] + off

    def mean3(ra, rb, rc, nf):
        def m_body(j, _):
            for f in range(nf):
                sl = pl.ds(f * 16, 16)
                ra[j, sl] = (ra[j, sl] + rb[j, sl] + rc[j, sl]) * (1.0 / 3.0)
            return 0
        lax.fori_loop(0, _BPW, m_body, 0)

    def rating_mean(out_ref):
        pltpu.async_copy(r0_h.at[idxv], ha, sem).wait()
        pltpu.async_copy(r1_h.at[idxv], hb, sem).wait()
        pltpu.async_copy(r2_h.at[idxv], hc, sem).wait()
        mean3(ha, hb, hc, 2)
        pltpu.sync_copy(ha, out_ref.at[sl_out])

    pltpu.sync_copy(users_h.at[sl_out], idxv)
    pltpu.async_copy(u1_h.at[idxv], rows_a, sem).wait()
    pltpu.async_copy(e1s_h.at[idxv], rows_b, sem).wait()
    pltpu.async_copy(e2s_h.at[idxv], rows_c, sem).wait()
    mean3(rows_a, rows_b, rows_c, 4)
    pltpu.sync_copy(rows_a, us_o.at[sl_out])

    rating_mean(ura_o)
    shift_idx(NN)
    rating_mean(urb_o)

    pltpu.sync_copy(pos_h.at[sl_out], idxv)
    pltpu.async_copy(it1_h.at[idxv], rows_a, sem).wait()
    pltpu.sync_copy(rows_a, ps_o.at[sl_out])
    shift_idx(NU)
    rating_mean(pra_o)
    shift_idx(NN)
    rating_mean(prb_o)

    pltpu.sync_copy(neg_h.at[sl_out], idxv)
    pltpu.async_copy(it1_h.at[idxv], rows_a, sem).wait()
    pltpu.sync_copy(rows_a, ns_o.at[sl_out])
    shift_idx(NU)
    rating_mean(nra_o)
    shift_idx(NN)
    rating_mean(nrb_o)


def _final(users, pos, neg, u1, e1s, e2s, it1, r0, r1, r2):
    mesh = plsc.VectorSubcoreMesh(core_axis_name="c", subcore_axis_name="s")
    o64 = jax.ShapeDtypeStruct((BS, H), jnp.float32)
    o32 = jax.ShapeDtypeStruct((BS, HH), jnp.float32)
    return pl.kernel(
        _final_body,
        out_type=(o64, o64, o64, o32, o32, o32, o32, o32, o32),
        mesh=mesh,
        scratch_types=[
            pltpu.VMEM((_BPW,), jnp.int32),
            pltpu.VMEM((_BPW, H), jnp.float32),
            pltpu.VMEM((_BPW, H), jnp.float32),
            pltpu.VMEM((_BPW, H), jnp.float32),
            pltpu.VMEM((_BPW, HH), jnp.float32),
            pltpu.VMEM((_BPW, HH), jnp.float32),
            pltpu.VMEM((_BPW, HH), jnp.float32),
            pltpu.SemaphoreType.DMA,
        ],
        compiler_params=pltpu.CompilerParams(use_tc_tiling_on_sc=False),
    )(users, pos, neg, u1, e1s, e2s, it1, r0, r1, r2)


def kernel(users, pos, neg, S, edge_index, edge_weight,
           user1_w, item1_w, user2_w, item2_w):
    users = users.astype(jnp.int32)
    pos = pos.astype(jnp.int32)
    neg_flat = neg.reshape(BS).astype(jnp.int32)

    npad = _EPAD - ET
    src = jnp.concatenate(
        [edge_index[0].astype(jnp.int32), jnp.zeros((npad,), jnp.int32)])
    dst = jnp.concatenate(
        [edge_index[1].astype(jnp.int32), jnp.zeros((npad,), jnp.int32)])
    w = jnp.concatenate(
        [edge_weight.astype(jnp.float32), jnp.zeros((npad,), jnp.float32)])
    dst3 = dst.reshape(_EPAD // _CK, _NSUB, _SUB)

    e1s = _matmul(S, user1_w)
    e2s = _matmul(S, e1s)

    zeros = jnp.zeros((_ZR, HH), jnp.float32)
    r0 = jnp.concatenate([user2_w[:, :HH], item2_w[:, :HH],
                          user2_w[:, HH:], item2_w[:, HH:]], axis=0)
    r1 = _hop(zeros, r0, src, dst3, w)
    r2 = _hop(zeros, r1, src, dst3, w)

    (us, ps, ns, ura, urb, pra, prb, nra, nrb) = _final(
        users, pos, neg_flat, user1_w, e1s, e2s, item1_w, r0, r1, r2)
    ur = jnp.concatenate([ura, urb], axis=1)
    pr = jnp.concatenate([pra, prb], axis=1)
    nr = jnp.concatenate([nra, nrb], axis=1)
    return (us, ps, ns.reshape(BS, 1, H),
            ur, pr, nr.reshape(BS, 1, H))


# speedup vs baseline: 1.4064x; 1.4064x over previous
import jax
import jax.numpy as jnp
from jax import lax
from jax.experimental import pallas as pl
from jax.experimental.pallas import tpu as pltpu
from jax.experimental.pallas import tpu_sc as plsc

NU = 10000
NI = 40000
NN = 50000
H = 64
HH = H // 2
ET = 800000
BS = 4096


_BM = 200


def _mm_body(s_ref, x_ref, o_ref):
    o_ref[...] = jnp.dot(s_ref[...], x_ref[...],
                         preferred_element_type=jnp.float32)


def _matmul(S, x):
    m, k = S.shape
    n = x.shape[1]
    return pl.pallas_call(
        _mm_body,
        grid=(m // _BM,),
        in_specs=[pl.BlockSpec((_BM, k), lambda i: (i, 0)),
                  pl.BlockSpec((k, n), lambda i: (0, 0))],
        out_specs=pl.BlockSpec((_BM, n), lambda i: (i, 0)),
        out_shape=jax.ShapeDtypeStruct((m, n), jnp.float32),
        compiler_params=pltpu.CompilerParams(
            dimension_semantics=("arbitrary",)),
    )(S, x)


_CK = 384
_SUB = 128
_NSUB = _CK // _SUB
_TCH = 132
_EPAD = 16 * _TCH * _CK
_ACC_ROWS = 50176
_ZR = _ACC_ROWS // 16
_WBR = NN // 16


def _hop_body(zeros_h, ein_h, src_h, dst3_h, w_h, eout_h,
              sv, dv, sdv, wv, rv, acc, isem0, isem1, gsem, ssem):
    c = lax.axis_index("c")
    s = lax.axis_index("s")
    shift = c * NN
    c0 = s * _TCH
    isems = (isem0, isem1)

    pltpu.sync_copy(zeros_h, acc.at[pl.ds(s * _ZR, _ZR)])
    plsc.subcore_barrier()

    def idx_copies(cid, b):
        base = (c0 + cid) * _CK
        return (
            pltpu.make_async_copy(src_h.at[pl.ds(base, _CK)], sv.at[b],
                                  isems[b]),
            pltpu.make_async_copy(dst3_h.at[c0 + cid], dv.at[b], isems[b]),
            pltpu.make_async_copy(w_h.at[pl.ds(base, _CK)], wv.at[b],
                                  isems[b]),
        )

    def shift_src(b):
        def sb(g, _):
            sl = pl.ds(g * 16, 16)
            sv[b, sl] = sv[b, sl] + shift
            return 0
        lax.fori_loop(0, _CK // 16, sb, 0)

    def gather_copies(b):
        return [pltpu.make_async_copy(
            ein_h.at[sv.at[b, pl.ds(j * _SUB, _SUB)]],
            rv.at[b, pl.ds(j * _SUB, _SUB)], gsem) for j in range(_NSUB)]

    def scatter_copies(b):
        return [pltpu.make_async_copy(
            rv.at[b, pl.ds(j * _SUB, _SUB)], acc.at[sdv.at[b, j]], ssem)
            for j in range(_NSUB)]

    def compute(b):
        def grp(g, _):
            wvec = wv[b, pl.ds(g * 16, 16)]
            for l in range(16):
                wj = wvec[l]
                for f in range(2):
                    fs = pl.ds(f * 16, 16)
                    rv[b, g * 16 + l, fs] = rv[b, g * 16 + l, fs] * wj
            return 0

        lax.fori_loop(0, _CK // 16, grp, 0)
        for j in range(_NSUB):
            def cpy(g, _):
                sl = pl.ds(g * 16, 16)
                sdv[b, j, sl] = dv[b, j, sl]
                return 0
            lax.fori_loop(0, _SUB // 16, cpy, 0)
        for d in scatter_copies(b):
            d.start(add=True)

    def step(cid, b):
        nb = 1 - b
        for d in gather_copies(b):
            d.wait()

        @pl.when(cid + 1 < _TCH)
        def _():
            for d in idx_copies(cid + 1, nb):
                d.wait()
            shift_src(nb)

        @pl.when(cid >= 1)
        def _():
            for d in scatter_copies(nb):
                d.wait()

        @pl.when(cid + 1 < _TCH)
        def _():
            for d in gather_copies(nb):
                d.start()

        compute(b)

        @pl.when(cid + 2 < _TCH)
        def _():
            for d in idx_copies(cid + 2, b):
                d.start()

    for d in idx_copies(0, 0):
        d.start()
    for d in idx_copies(0, 0):
        d.wait()
    shift_src(0)
    for d in gather_copies(0):
        d.start()
    for d in idx_copies(1, 1):
        d.start()

    def pair(i2, _):
        step(2 * i2, 0)
        step(2 * i2 + 1, 1)
        return 0

    lax.fori_loop(0, _TCH // 2, pair, 0)
    for d in scatter_copies(1):
        d.wait()

    plsc.subcore_barrier()

    pltpu.sync_copy(acc.at[pl.ds(s * _WBR, _WBR)],
                    eout_h.at[pl.ds(c * NN + s * _WBR, _WBR)])


def _hop(zeros, ein, src, dst3, w):
    mesh = plsc.VectorSubcoreMesh(core_axis_name="c", subcore_axis_name="s")
    return pl.kernel(
        _hop_body,
        out_type=jax.ShapeDtypeStruct((2 * NN, HH), jnp.float32),
        mesh=mesh,
        scratch_types=[
            pltpu.VMEM((2, _CK), jnp.int32),
            pltpu.VMEM((2, _NSUB, _SUB), jnp.int32),
            pltpu.VMEM((2, _NSUB, _SUB), jnp.int32),
            pltpu.VMEM((2, _CK), jnp.float32),
            pltpu.VMEM((2, _CK, HH), jnp.float32),
            pltpu.VMEM_SHARED((_ACC_ROWS, HH), jnp.float32),
            pltpu.SemaphoreType.DMA,
            pltpu.SemaphoreType.DMA,
            pltpu.SemaphoreType.DMA,
            pltpu.SemaphoreType.DMA,
        ],
        compiler_params=pltpu.CompilerParams(use_tc_tiling_on_sc=False),
    )(zeros, ein, src, dst3, w)


_BPW = BS // 32


def _final_body(users_h, pos_h, neg_h, u1_h, e1s_h, e2s_h, it1_h,
                r0_h, r1_h, r2_h,
                us_o, ps_o, ns_o, ura_o, urb_o, pra_o, prb_o, nra_o, nrb_o,
                idxv, rows_a, rows_b, rows_c, ha, hb, hc, sem):
    c = lax.axis_index("c")
    s = lax.axis_index("s")
    w = s * 2 + c
    base = w * _BPW
    sl_out = pl.ds(base, _BPW)

    def shift_idx(off):
        for g in range(_BPW // 16):
            sl = pl.ds(g * 16, 16)
            idxv[sl] = idxv[sl] + off

    def mean3(ra, rb, rc, nf):
        def m_body(j, _):
            for f in range(nf):
                sl = pl.ds(f * 16, 16)
                ra[j, sl] = (ra[j, sl] + rb[j, sl] + rc[j, sl]) * (1.0 / 3.0)
            return 0
        lax.fori_loop(0, _BPW, m_body, 0)

    def rating_mean(out_ref):
        pltpu.async_copy(r0_h.at[idxv], ha, sem).wait()
        pltpu.async_copy(r1_h.at[idxv], hb, sem).wait()
        pltpu.async_copy(r2_h.at[idxv], hc, sem).wait()
        mean3(ha, hb, hc, 2)
        pltpu.sync_copy(ha, out_ref.at[sl_out])

    pltpu.sync_copy(users_h.at[sl_out], idxv)
    pltpu.async_copy(u1_h.at[idxv], rows_a, sem).wait()
    pltpu.async_copy(e1s_h.at[idxv], rows_b, sem).wait()
    pltpu.async_copy(e2s_h.at[idxv], rows_c, sem).wait()
    mean3(rows_a, rows_b, rows_c, 4)
    pltpu.sync_copy(rows_a, us_o.at[sl_out])

    rating_mean(ura_o)
    shift_idx(NN)
    rating_mean(urb_o)

    pltpu.sync_copy(pos_h.at[sl_out], idxv)
    pltpu.async_copy(it1_h.at[idxv], rows_a, sem).wait()
    pltpu.sync_copy(rows_a, ps_o.at[sl_out])
    shift_idx(NU)
    rating_mean(pra_o)
    shift_idx(NN)
    rating_mean(prb_o)

    pltpu.sync_copy(neg_h.at[sl_out], idxv)
    pltpu.async_copy(it1_h.at[idxv], rows_a, sem).wait()
    pltpu.sync_copy(rows_a, ns_o.at[sl_out])
    shift_idx(NU)
    rating_mean(nra_o)
    shift_idx(NN)
    rating_mean(nrb_o)


def _final(users, pos, neg, u1, e1s, e2s, it1, r0, r1, r2):
    mesh = plsc.VectorSubcoreMesh(core_axis_name="c", subcore_axis_name="s")
    o64 = jax.ShapeDtypeStruct((BS, H), jnp.float32)
    o32 = jax.ShapeDtypeStruct((BS, HH), jnp.float32)
    return pl.kernel(
        _final_body,
        out_type=(o64, o64, o64, o32, o32, o32, o32, o32, o32),
        mesh=mesh,
        scratch_types=[
            pltpu.VMEM((_BPW,), jnp.int32),
            pltpu.VMEM((_BPW, H), jnp.float32),
            pltpu.VMEM((_BPW, H), jnp.float32),
            pltpu.VMEM((_BPW, H), jnp.float32),
            pltpu.VMEM((_BPW, HH), jnp.float32),
            pltpu.VMEM((_BPW, HH), jnp.float32),
            pltpu.VMEM((_BPW, HH), jnp.float32),
            pltpu.SemaphoreType.DMA,
        ],
        compiler_params=pltpu.CompilerParams(use_tc_tiling_on_sc=False),
    )(users, pos, neg, u1, e1s, e2s, it1, r0, r1, r2)


def kernel(users, pos, neg, S, edge_index, edge_weight,
           user1_w, item1_w, user2_w, item2_w):
    users = users.astype(jnp.int32)
    pos = pos.astype(jnp.int32)
    neg_flat = neg.reshape(BS).astype(jnp.int32)

    npad = _EPAD - ET
    src = jnp.concatenate(
        [edge_index[0].astype(jnp.int32), jnp.zeros((npad,), jnp.int32)])
    dst = jnp.concatenate(
        [edge_index[1].astype(jnp.int32), jnp.zeros((npad,), jnp.int32)])
    w = jnp.concatenate(
        [edge_weight.astype(jnp.float32), jnp.zeros((npad,), jnp.float32)])
    dst3 = dst.reshape(_EPAD // _CK, _NSUB, _SUB)

    e1s = _matmul(S, user1_w)
    e2s = _matmul(S, e1s)

    zeros = jnp.zeros((_ZR, HH), jnp.float32)
    r0 = jnp.concatenate([user2_w[:, :HH], item2_w[:, :HH],
                          user2_w[:, HH:], item2_w[:, HH:]], axis=0)
    r1 = _hop(zeros, r0, src, dst3, w)
    r2 = _hop(zeros, r1, src, dst3, w)

    (us, ps, ns, ura, urb, pra, prb, nra, nrb) = _final(
        users, pos, neg_flat, user1_w, e1s, e2s, item1_w, r0, r1, r2)
    ur = jnp.concatenate([ura, urb], axis=1)
    pr = jnp.concatenate([pra, prb], axis=1)
    nr = jnp.concatenate([nra, nrb], axis=1)
    return (us, ps, ns.reshape(BS, 1, H),
            ur, pr, nr.reshape(BS, 1, H))
